# baseline (speedup 1.0000x reference)
import functools

import jax
import jax.numpy as jnp
from jax import lax
from jax.experimental import pallas as pl
from jax.experimental.pallas import tpu as pltpu
from jax.experimental.pallas import tpu_sc as plsc

N = 16384
K = 1000
NC = 2
NS = 16
NW = NC * NS
SB = 4096
CB = 8
NQ = N // SB
NU = (K // CB) * NQ
UPW = 16
L = 16


def _scan_scatter(x_v, buf, c0, s0, val, iota):
    def body(j, carry):
        xv = x_v[pl.ds(s0 + j * L, L)]
        rows = xv - c0
        mask = (xv >= c0) & (xv < c0 + CB)
        plsc.store_scatter(buf, [rows, iota + j * L], val, mask=mask)
        return carry

    lax.fori_loop(0, SB // L, body, 0, unroll=4)


def _onehot_body(x_hbm, z_hbm, out_hbm, x_v, buf_a, buf_b, sem_a, sem_b):
    wid = lax.axis_index("s") * NC + lax.axis_index("c")

    pltpu.sync_copy(x_hbm, x_v)
    za = pltpu.async_copy(z_hbm, buf_a, sem_a)
    zb = pltpu.async_copy(z_hbm, buf_b, sem_b)
    za.wait()
    zb.wait()

    zeros = jnp.zeros((L,), jnp.int32)
    ones = jnp.ones((L,), jnp.int32)
    iota = lax.iota(jnp.int32, L)

    def unit_coords(u):
        rb = u // NQ
        q = u - rb * NQ
        return rb * CB, q * SB

    def do_unit(i, buf, sem):
        u = wid + NW * i
        c0, s0 = unit_coords(u)
        if i >= 2:
            pltpu.make_async_copy(
                buf, out_hbm.at[pl.ds(0, CB), pl.ds(0, SB)], sem
            ).wait()
            pc0, ps0 = unit_coords(u - 2 * NW)
            _scan_scatter(x_v, buf, pc0, ps0, zeros, iota)
        _scan_scatter(x_v, buf, c0, s0, ones, iota)
        pltpu.async_copy(
            buf, out_hbm.at[pl.ds(c0, CB), pl.ds(s0, SB)], sem
        )

    for i in range(UPW - 1):
        buf, sem = (buf_a, sem_a) if i % 2 == 0 else (buf_b, sem_b)
        do_unit(i, buf, sem)

    last = UPW - 1
    lbuf, lsem = (buf_a, sem_a) if last % 2 == 0 else (buf_b, sem_b)

    @pl.when(wid + NW * last < NU)
    def _():
        do_unit(last, lbuf, lsem)

    pltpu.make_async_copy(
        buf_a if (UPW - 2) % 2 == 0 else buf_b,
        out_hbm.at[pl.ds(0, CB), pl.ds(0, SB)],
        sem_a if (UPW - 2) % 2 == 0 else sem_b,
    ).wait()

    @pl.when(wid + NW * last < NU)
    def _():
        pltpu.make_async_copy(
            lbuf, out_hbm.at[pl.ds(0, CB), pl.ds(0, SB)], lsem
        ).wait()


@jax.jit
def kernel(x):
    run = functools.partial(
        pl.kernel,
        out_type=jax.ShapeDtypeStruct((K, N), jnp.int32),
        mesh=plsc.VectorSubcoreMesh(core_axis_name="c", subcore_axis_name="s"),
        compiler_params=pltpu.CompilerParams(needs_layout_passes=False),
        scratch_types=[
            pltpu.VMEM((N,), jnp.int32),
            pltpu.VMEM((CB, SB), jnp.int32),
            pltpu.VMEM((CB, SB), jnp.int32),
            pltpu.SemaphoreType.DMA,
            pltpu.SemaphoreType.DMA,
        ],
    )(_onehot_body)
    zeros_unit = jnp.zeros((CB, SB), jnp.int32)
    out_t = run(x, zeros_unit)
    return out_t.T


# speedup vs baseline: 1.9145x; 1.9145x over previous
import functools

import jax
import jax.numpy as jnp
from jax import lax
from jax.experimental import pallas as pl
from jax.experimental.pallas import tpu as pltpu
from jax.experimental.pallas import tpu_sc as plsc

N = 16384
K = 1000
NC = 2
NS = 16
NW = NC * NS
SPW = N // NW
CC = 200
NCHUNK = K // CC
L = 16


def _onehot_body(x_hbm, z_hbm, out_hbm, x_v, buf):
    wid = lax.axis_index("s") * NC + lax.axis_index("c")
    base = wid * SPW

    pltpu.sync_copy(x_hbm.at[pl.ds(base, SPW)], x_v)
    pltpu.sync_copy(z_hbm, buf)

    zeros = jnp.zeros((L,), jnp.int32)
    ones = jnp.ones((L,), jnp.int32)
    iota = lax.iota(jnp.int32, L)

    for chunk in range(NCHUNK):
        c0 = chunk * CC
        for j in range(SPW // L):
            xv = x_v[pl.ds(j * L, L)]
            rows = xv - c0
            mask = (xv >= c0) & (xv < c0 + CC)
            plsc.store_scatter(buf, [rows, iota + j * L], ones, mask=mask)
        pltpu.sync_copy(buf, out_hbm.at[pl.ds(c0, CC), pl.ds(base, SPW)])
        if chunk + 1 < NCHUNK:
            for j in range(SPW // L):
                xv = x_v[pl.ds(j * L, L)]
                rows = xv - c0
                mask = (xv >= c0) & (xv < c0 + CC)
                plsc.store_scatter(buf, [rows, iota + j * L], zeros, mask=mask)


@jax.jit
def kernel(x):
    run = functools.partial(
        pl.kernel,
        out_type=jax.ShapeDtypeStruct((K, N), jnp.int32),
        mesh=plsc.VectorSubcoreMesh(core_axis_name="c", subcore_axis_name="s"),
        compiler_params=pltpu.CompilerParams(needs_layout_passes=False),
        scratch_types=[
            pltpu.VMEM((SPW,), jnp.int32),
            pltpu.VMEM((CC, SPW), jnp.int32),
        ],
    )(_onehot_body)
    zeros_chunk = jnp.zeros((CC, SPW), jnp.int32)
    out_t = run(x, zeros_chunk)
    return out_t.T
